# baseline (speedup 1.0000x reference)
import functools

import jax
import jax.numpy as jnp
from jax import lax
from jax.experimental import pallas as pl
from jax.experimental.pallas import tpu as pltpu
from jax.experimental.pallas import tpu_sc as plsc

F = 128
NC = 2
NS = 16
NW = NC * NS
EB = 128

_sc_mesh = functools.partial(
    plsc.VectorSubcoreMesh,
    core_axis_name="c", subcore_axis_name="s", num_cores=NC, num_subcores=NS,
)


def _deg_body(col_hbm, ew_hbm, out_hbm, col_v, ew_v, zb_v, acc_s):
    c = lax.axis_index("c")
    s = lax.axis_index("s")
    wid = s * NC + c
    n_pad = acc_s.shape[0]
    nb = col_v.shape[0]
    rows_per_tile = n_pad // NS
    pltpu.sync_copy(col_hbm.at[pl.ds(wid * nb, nb)], col_v)
    pltpu.sync_copy(ew_hbm.at[pl.ds(wid * nb, nb)], ew_v)

    def zero(i, carry):
        zb_v[pl.ds(i * 16, 16)] = jnp.zeros((16,), jnp.float32)
        return carry

    lax.fori_loop(0, rows_per_tile // 16, zero, 0)
    zbase = s * rows_per_tile
    pltpu.sync_copy(zb_v, acc_s.at[pl.ds(zbase, rows_per_tile)])
    plsc.subcore_barrier()

    def body(j, carry):
        pltpu.sync_copy(ew_v.at[j], acc_s.at[col_v.at[j]], add=True)
        return carry

    lax.fori_loop(0, nb, body, 0)
    plsc.subcore_barrier()
    pltpu.sync_copy(acc_s.at[pl.ds(zbase, rows_per_tile)],
                    out_hbm.at[pl.ds(c * n_pad + zbase, rows_per_tile)])


def _spmm_body(hp_hbm, row_hbm, col_hbm, ew_hbm, zeros_hbm, out_hbm,
               row_v, col_v, ew_v, gbuf, acc_s, sem):
    c = lax.axis_index("c")
    s = lax.axis_index("s")
    wid = s * NC + c
    nb = row_v.shape[0]
    n_pad = acc_s.shape[0]
    rows_per_tile = n_pad // NS
    pltpu.sync_copy(row_hbm.at[pl.ds(wid * nb, nb)], row_v)
    pltpu.sync_copy(col_hbm.at[pl.ds(wid * nb, nb)], col_v)
    pltpu.sync_copy(ew_hbm.at[pl.ds(wid * nb, nb)], ew_v)
    zbase = s * rows_per_tile
    pltpu.sync_copy(zeros_hbm, acc_s.at[pl.ds(zbase, rows_per_tile)])
    plsc.subcore_barrier()

    def batch(j, carry):
        pltpu.async_copy(hp_hbm.at[row_v.at[j]], gbuf, sem).wait()

        def grp(t, carry2):
            wvec = ew_v[j, pl.ds(t * 16, 16)]
            base = t * 16
            for l in range(16):
                wv = jnp.full((16,), wvec[l], jnp.float32)
                for q in range(F // 16):
                    sl = pl.ds(q * 16, 16)
                    gbuf[base + l, sl] = gbuf[base + l, sl] * wv
            return carry2

        lax.fori_loop(0, EB // 16, grp, 0)
        pltpu.sync_copy(gbuf, acc_s.at[col_v.at[j]], add=True)
        return carry

    lax.fori_loop(0, nb, batch, 0)
    plsc.subcore_barrier()
    pltpu.sync_copy(acc_s.at[pl.ds(zbase, rows_per_tile)],
                    out_hbm.at[pl.ds(c * n_pad + zbase, rows_per_tile)])


def _tc1_body(x_ref, w1_ref, dega_ref, degb_ref, hp_ref, dis_ref):
    deg = dega_ref[...] + degb_ref[...] + 1.0
    dis = jnp.where(deg > 0, lax.rsqrt(deg), 0.0)
    h = jnp.dot(x_ref[...], w1_ref[...], preferred_element_type=jnp.float32)
    hp_ref[...] = h * dis[:, None]
    dis_ref[...] = dis


def _tc2_body(s1a_ref, s1b_ref, hp1_ref, dis_ref, b1_ref, w2_ref, hp2_ref):
    dis = dis_ref[...]
    t = dis[:, None] * (s1a_ref[...] + s1b_ref[...] + hp1_ref[...])
    t = t + b1_ref[...][None, :]
    o = jnp.maximum(t, 0.0)
    h2 = jnp.dot(o, w2_ref[...], preferred_element_type=jnp.float32)
    hp2_ref[...] = h2 * dis[:, None]


def _tc3_body(s2a_ref, s2b_ref, hp2_ref, dis_ref, b2_ref, out_ref):
    dis = dis_ref[...]
    t = dis[:, None] * (s2a_ref[...] + s2b_ref[...] + hp2_ref[...])
    t = t + b2_ref[...][None, :]
    m = jnp.max(t, axis=1, keepdims=True)
    lse = m + jnp.log(jnp.sum(jnp.exp(t - m), axis=1, keepdims=True))
    out_ref[...] = t - lse


def kernel(x, edge_index, edge_weight, W1, b1, W2, b2):
    n, f = x.shape
    e = edge_index.shape[1]
    n_pad = ((n + NW * 8 - 1) // (NW * 8)) * (NW * 8)
    e_quant = NW * EB * 8
    e_pad = ((e + e_quant - 1) // e_quant) * e_quant
    nb_tile = e_pad // (NW * EB)
    rows_per_tile = n_pad // NS

    row = edge_index[0].astype(jnp.int32)
    col = edge_index[1].astype(jnp.int32)
    ew = edge_weight.astype(jnp.float32)
    zi = jnp.zeros((e_pad - e,), jnp.int32)
    row_p = jnp.concatenate([row, zi])
    col_p = jnp.concatenate([col, zi])
    ew_p = jnp.concatenate([ew, jnp.zeros((e_pad - e,), jnp.float32)])
    row2d = row_p.reshape(e_pad // EB, EB)
    col2d = col_p.reshape(e_pad // EB, EB)
    ew2d = ew_p.reshape(e_pad // EB, EB)
    x_p = jnp.concatenate([x, jnp.zeros((n_pad - n, f), x.dtype)], axis=0)
    zeros_rows = jnp.zeros((rows_per_tile, F), jnp.float32)

    deg_k = pl.kernel(
        _deg_body,
        out_type=jax.ShapeDtypeStruct((NC * n_pad,), jnp.float32),
        mesh=_sc_mesh(),
        scratch_types=[
            pltpu.VMEM((nb_tile, EB), jnp.int32),
            pltpu.VMEM((nb_tile, EB), jnp.float32),
            pltpu.VMEM((rows_per_tile,), jnp.float32),
            pltpu.VMEM_SHARED((n_pad,), jnp.float32),
        ],
    )
    deg2 = deg_k(col2d, ew2d)
    deg_a, deg_b = deg2[:n_pad], deg2[n_pad:]

    spmm_k = pl.kernel(
        _spmm_body,
        out_type=jax.ShapeDtypeStruct((NC * n_pad, F), jnp.float32),
        mesh=_sc_mesh(),
        scratch_types=[
            pltpu.VMEM((nb_tile, EB), jnp.int32),
            pltpu.VMEM((nb_tile, EB), jnp.int32),
            pltpu.VMEM((nb_tile, EB), jnp.float32),
            pltpu.VMEM((EB, F), jnp.float32),
            pltpu.VMEM_SHARED((n_pad, F), jnp.float32),
            pltpu.SemaphoreType.DMA,
        ],
    )

    def spmm(hp):
        sh = spmm_k(hp, row2d, col2d, ew2d, zeros_rows)
        return sh[:n_pad], sh[n_pad:]

    blk = 1024
    grid = (n_pad // blk,)
    hp1, dis = pl.pallas_call(
        _tc1_body,
        grid=grid,
        in_specs=[
            pl.BlockSpec((blk, f), lambda i: (i, 0)),
            pl.BlockSpec((f, F), lambda i: (0, 0)),
            pl.BlockSpec((blk,), lambda i: (i,)),
            pl.BlockSpec((blk,), lambda i: (i,)),
        ],
        out_specs=[
            pl.BlockSpec((blk, F), lambda i: (i, 0)),
            pl.BlockSpec((blk,), lambda i: (i,)),
        ],
        out_shape=[
            jax.ShapeDtypeStruct((n_pad, F), jnp.float32),
            jax.ShapeDtypeStruct((n_pad,), jnp.float32),
        ],
    )(x_p, W1, deg_a, deg_b)

    s1a, s1b = spmm(hp1)

    hp2 = pl.pallas_call(
        _tc2_body,
        grid=grid,
        in_specs=[
            pl.BlockSpec((blk, F), lambda i: (i, 0)),
            pl.BlockSpec((blk, F), lambda i: (i, 0)),
            pl.BlockSpec((blk, F), lambda i: (i, 0)),
            pl.BlockSpec((blk,), lambda i: (i,)),
            pl.BlockSpec((F,), lambda i: (0,)),
            pl.BlockSpec((F, F), lambda i: (0, 0)),
        ],
        out_specs=pl.BlockSpec((blk, F), lambda i: (i, 0)),
        out_shape=jax.ShapeDtypeStruct((n_pad, F), jnp.float32),
    )(s1a, s1b, hp1, dis, b1, W2)

    s2a, s2b = spmm(hp2)

    out = pl.pallas_call(
        _tc3_body,
        grid=grid,
        in_specs=[
            pl.BlockSpec((blk, F), lambda i: (i, 0)),
            pl.BlockSpec((blk, F), lambda i: (i, 0)),
            pl.BlockSpec((blk, F), lambda i: (i, 0)),
            pl.BlockSpec((blk,), lambda i: (i,)),
            pl.BlockSpec((F,), lambda i: (0,)),
        ],
        out_specs=pl.BlockSpec((blk, F), lambda i: (i, 0)),
        out_shape=jax.ShapeDtypeStruct((n_pad, F), jnp.float32),
    )(s2a, s2b, hp2, dis, b2)

    return out[:n]


# speedup vs baseline: 1.1614x; 1.1614x over previous
import functools

import jax
import jax.numpy as jnp
from jax import lax
from jax.experimental import pallas as pl
from jax.experimental.pallas import tpu as pltpu
from jax.experimental.pallas import tpu_sc as plsc

F = 128
NC = 2
NS = 16
NW = NC * NS
EB = 128

_sc_mesh = functools.partial(
    plsc.VectorSubcoreMesh,
    core_axis_name="c", subcore_axis_name="s", num_cores=NC, num_subcores=NS,
)


def _deg_body(col_hbm, ew_hbm, out_hbm, col_v, ew_v, zb_v, acc_s):
    c = lax.axis_index("c")
    s = lax.axis_index("s")
    wid = s * NC + c
    n_pad = acc_s.shape[0]
    nb = col_v.shape[0]
    rows_per_tile = n_pad // NS
    pltpu.sync_copy(col_hbm.at[pl.ds(wid * nb, nb)], col_v)
    pltpu.sync_copy(ew_hbm.at[pl.ds(wid * nb, nb)], ew_v)

    def zero(i, carry):
        zb_v[pl.ds(i * 16, 16)] = jnp.zeros((16,), jnp.float32)
        return carry

    lax.fori_loop(0, rows_per_tile // 16, zero, 0)
    zbase = s * rows_per_tile
    pltpu.sync_copy(zb_v, acc_s.at[pl.ds(zbase, rows_per_tile)])
    plsc.subcore_barrier()

    def body(j, carry):
        pltpu.sync_copy(ew_v.at[j], acc_s.at[col_v.at[j]], add=True)
        return carry

    lax.fori_loop(0, nb, body, 0)
    plsc.subcore_barrier()
    pltpu.sync_copy(acc_s.at[pl.ds(zbase, rows_per_tile)],
                    out_hbm.at[pl.ds(c * n_pad + zbase, rows_per_tile)])


def _make_spmm_body(nb_tile, nchunk):
    ch = nb_tile // nchunk

    def _spmm_body(hp_hbm, row_hbm, col_hbm, ew_hbm, zeros_hbm, out_hbm,
                   row_c, col_c, ew_c, gbuf0, gbuf1, acc_s, *sems):
        c = lax.axis_index("c")
        s = lax.axis_index("s")
        wid = s * NC + c
        n_pad = acc_s.shape[0]
        rows_per_tile = n_pad // NS
        ebase = wid * nb_tile
        gbufs = (gbuf0, gbuf1)
        sg = sems[:2]
        se = sems[2:]

        def chunk_copies(ci, slot):
            return (
                (row_hbm.at[pl.ds(ebase + ci * ch, ch)], row_c.at[slot]),
                (col_hbm.at[pl.ds(ebase + ci * ch, ch)], col_c.at[slot]),
                (ew_hbm.at[pl.ds(ebase + ci * ch, ch)], ew_c.at[slot]),
            )

        def start_gather(slot, k, b):
            pltpu.async_copy(hp_hbm.at[row_c.at[slot, k]], gbufs[b], sg[b])

        def wait_gather(slot, k, b):
            pltpu.make_async_copy(
                hp_hbm.at[row_c.at[slot, k]], gbufs[b], sg[b]).wait()

        def scale(slot, k, b):
            def grp(t, carry2):
                wvec = ew_c[slot, k, pl.ds(t * 16, 16)]
                base = t * 16
                for l in range(16):
                    wv = jnp.full((16,), wvec[l], jnp.float32)
                    for q in range(F // 16):
                        sl = pl.ds(q * 16, 16)
                        gbufs[b][base + l, sl] = gbufs[b][base + l, sl] * wv
                return carry2

            lax.fori_loop(0, EB // 16, grp, 0)

        for src_ref, dst_ref in chunk_copies(0, 0):
            pltpu.sync_copy(src_ref, dst_ref)
        zbase = s * rows_per_tile
        pltpu.sync_copy(zeros_hbm, acc_s.at[pl.ds(zbase, rows_per_tile)])
        plsc.subcore_barrier()

        for ci in range(nchunk):
            slot = ci % 2
            if ci > 0:
                for src_ref, dst_ref in chunk_copies(ci, slot):
                    pltpu.make_async_copy(src_ref, dst_ref, se[slot]).wait()
            if ci + 1 < nchunk:
                for src_ref, dst_ref in chunk_copies(ci + 1, 1 - slot):
                    pltpu.async_copy(src_ref, dst_ref, se[1 - slot])

            start_gather(slot, 0, 0)

            def pair(i, carry):
                for b in range(2):
                    k = i * 2 + b
                    wait_gather(slot, k, b)

                    @pl.when(k + 1 < ch)
                    def _():
                        start_gather(slot, k + 1, 1 - b)

                    scale(slot, k, b)
                    pltpu.sync_copy(gbufs[b], acc_s.at[col_c.at[slot, k]],
                                    add=True)
                return carry

            lax.fori_loop(0, ch // 2, pair, 0)

        plsc.subcore_barrier()
        pltpu.sync_copy(acc_s.at[pl.ds(zbase, rows_per_tile)],
                        out_hbm.at[pl.ds(c * n_pad + zbase, rows_per_tile)])

    return _spmm_body


def _tc1_body(x_ref, w1_ref, dega_ref, degb_ref, hp_ref, dis_ref):
    deg = dega_ref[...] + degb_ref[...] + 1.0
    dis = jnp.where(deg > 0, lax.rsqrt(deg), 0.0)
    h = jnp.dot(x_ref[...], w1_ref[...], preferred_element_type=jnp.float32)
    hp_ref[...] = h * dis[:, None]
    dis_ref[...] = dis


def _tc2_body(s1a_ref, s1b_ref, hp1_ref, dis_ref, b1_ref, w2_ref, hp2_ref):
    dis = dis_ref[...]
    t = dis[:, None] * (s1a_ref[...] + s1b_ref[...] + hp1_ref[...])
    t = t + b1_ref[...][None, :]
    o = jnp.maximum(t, 0.0)
    h2 = jnp.dot(o, w2_ref[...], preferred_element_type=jnp.float32)
    hp2_ref[...] = h2 * dis[:, None]


def _tc3_body(s2a_ref, s2b_ref, hp2_ref, dis_ref, b2_ref, out_ref):
    dis = dis_ref[...]
    t = dis[:, None] * (s2a_ref[...] + s2b_ref[...] + hp2_ref[...])
    t = t + b2_ref[...][None, :]
    m = jnp.max(t, axis=1, keepdims=True)
    lse = m + jnp.log(jnp.sum(jnp.exp(t - m), axis=1, keepdims=True))
    out_ref[...] = t - lse


def kernel(x, edge_index, edge_weight, W1, b1, W2, b2):
    n, f = x.shape
    e = edge_index.shape[1]
    n_pad = ((n + NW * 8 - 1) // (NW * 8)) * (NW * 8)
    e_quant = NW * EB * 8
    e_pad = ((e + e_quant - 1) // e_quant) * e_quant
    nb_tile = e_pad // (NW * EB)
    rows_per_tile = n_pad // NS

    row = edge_index[0].astype(jnp.int32)
    col = edge_index[1].astype(jnp.int32)
    ew = edge_weight.astype(jnp.float32)
    zi = jnp.zeros((e_pad - e,), jnp.int32)
    row_p = jnp.concatenate([row, zi])
    col_p = jnp.concatenate([col, zi])
    ew_p = jnp.concatenate([ew, jnp.zeros((e_pad - e,), jnp.float32)])
    row2d = row_p.reshape(e_pad // EB, EB)
    col2d = col_p.reshape(e_pad // EB, EB)
    ew2d = ew_p.reshape(e_pad // EB, EB)
    x_p = jnp.concatenate([x, jnp.zeros((n_pad - n, f), x.dtype)], axis=0)
    zeros_rows = jnp.zeros((rows_per_tile, F), jnp.float32)

    deg_k = pl.kernel(
        _deg_body,
        out_type=jax.ShapeDtypeStruct((NC * n_pad,), jnp.float32),
        mesh=_sc_mesh(),
        scratch_types=[
            pltpu.VMEM((nb_tile, EB), jnp.int32),
            pltpu.VMEM((nb_tile, EB), jnp.float32),
            pltpu.VMEM((rows_per_tile,), jnp.float32),
            pltpu.VMEM_SHARED((n_pad,), jnp.float32),
        ],
    )
    deg2 = deg_k(col2d, ew2d)
    deg_a, deg_b = deg2[:n_pad], deg2[n_pad:]

    nchunk = 5
    ch = nb_tile // nchunk
    spmm_k = pl.kernel(
        _make_spmm_body(nb_tile, nchunk),
        out_type=jax.ShapeDtypeStruct((NC * n_pad, F), jnp.float32),
        mesh=_sc_mesh(),
        scratch_types=[
            pltpu.VMEM((2, ch, EB), jnp.int32),
            pltpu.VMEM((2, ch, EB), jnp.int32),
            pltpu.VMEM((2, ch, EB), jnp.float32),
            pltpu.VMEM((EB, F), jnp.float32),
            pltpu.VMEM((EB, F), jnp.float32),
            pltpu.VMEM_SHARED((n_pad, F), jnp.float32),
        ] + [pltpu.SemaphoreType.DMA] * 4,
    )

    def spmm(hp):
        sh = spmm_k(hp, row2d, col2d, ew2d, zeros_rows)
        return sh[:n_pad], sh[n_pad:]

    blk = 1024
    grid = (n_pad // blk,)
    hp1, dis = pl.pallas_call(
        _tc1_body,
        grid=grid,
        in_specs=[
            pl.BlockSpec((blk, f), lambda i: (i, 0)),
            pl.BlockSpec((f, F), lambda i: (0, 0)),
            pl.BlockSpec((blk,), lambda i: (i,)),
            pl.BlockSpec((blk,), lambda i: (i,)),
        ],
        out_specs=[
            pl.BlockSpec((blk, F), lambda i: (i, 0)),
            pl.BlockSpec((blk,), lambda i: (i,)),
        ],
        out_shape=[
            jax.ShapeDtypeStruct((n_pad, F), jnp.float32),
            jax.ShapeDtypeStruct((n_pad,), jnp.float32),
        ],
    )(x_p, W1, deg_a, deg_b)

    s1a, s1b = spmm(hp1)

    hp2 = pl.pallas_call(
        _tc2_body,
        grid=grid,
        in_specs=[
            pl.BlockSpec((blk, F), lambda i: (i, 0)),
            pl.BlockSpec((blk, F), lambda i: (i, 0)),
            pl.BlockSpec((blk, F), lambda i: (i, 0)),
            pl.BlockSpec((blk,), lambda i: (i,)),
            pl.BlockSpec((F,), lambda i: (0,)),
            pl.BlockSpec((F, F), lambda i: (0, 0)),
        ],
        out_specs=pl.BlockSpec((blk, F), lambda i: (i, 0)),
        out_shape=jax.ShapeDtypeStruct((n_pad, F), jnp.float32),
    )(s1a, s1b, hp1, dis, b1, W2)

    s2a, s2b = spmm(hp2)

    out = pl.pallas_call(
        _tc3_body,
        grid=grid,
        in_specs=[
            pl.BlockSpec((blk, F), lambda i: (i, 0)),
            pl.BlockSpec((blk, F), lambda i: (i, 0)),
            pl.BlockSpec((blk, F), lambda i: (i, 0)),
            pl.BlockSpec((blk,), lambda i: (i,)),
            pl.BlockSpec((F,), lambda i: (0,)),
        ],
        out_specs=pl.BlockSpec((blk, F), lambda i: (i, 0)),
        out_shape=jax.ShapeDtypeStruct((n_pad, F), jnp.float32),
    )(s2a, s2b, hp2, dis, b2)

    return out[:n]


# speedup vs baseline: 1.1801x; 1.0161x over previous
import functools

import jax
import jax.numpy as jnp
from jax import lax
from jax.experimental import pallas as pl
from jax.experimental.pallas import tpu as pltpu
from jax.experimental.pallas import tpu_sc as plsc

F = 128
NC = 2
NS = 16
NW = NC * NS
EB = 128

_sc_mesh = functools.partial(
    plsc.VectorSubcoreMesh,
    core_axis_name="c", subcore_axis_name="s", num_cores=NC, num_subcores=NS,
)


def _deg_body(col_hbm, ew_hbm, out_hbm, col_v, ew_v, zb_v, acc_s):
    c = lax.axis_index("c")
    s = lax.axis_index("s")
    wid = s * NC + c
    n_pad = acc_s.shape[0]
    nb = col_v.shape[0]
    rows_per_tile = n_pad // NS
    pltpu.sync_copy(col_hbm.at[pl.ds(wid * nb, nb)], col_v)
    pltpu.sync_copy(ew_hbm.at[pl.ds(wid * nb, nb)], ew_v)

    def zero(i, carry):
        zb_v[pl.ds(i * 16, 16)] = jnp.zeros((16,), jnp.float32)
        return carry

    lax.fori_loop(0, rows_per_tile // 16, zero, 0)
    zbase = s * rows_per_tile
    pltpu.sync_copy(zb_v, acc_s.at[pl.ds(zbase, rows_per_tile)])
    plsc.subcore_barrier()

    def body(j, carry):
        pltpu.sync_copy(ew_v.at[j], acc_s.at[col_v.at[j]], add=True)
        return carry

    lax.fori_loop(0, nb, body, 0)
    plsc.subcore_barrier()
    pltpu.sync_copy(acc_s.at[pl.ds(zbase, rows_per_tile)],
                    out_hbm.at[pl.ds(c * n_pad + zbase, rows_per_tile)])


def _make_spmm_body(nb_tile, nchunk):
    ch = nb_tile // nchunk

    def _spmm_body(hp_hbm, row_hbm, col_hbm, ew_hbm, zeros_hbm, out_hbm,
                   row_c, col_c, ew_c, gbuf0, gbuf1, acc_s, *sems):
        c = lax.axis_index("c")
        s = lax.axis_index("s")
        wid = s * NC + c
        n_pad = acc_s.shape[0]
        rows_per_tile = n_pad // NS
        ebase = wid * nb_tile
        gbufs = (gbuf0, gbuf1)
        sg = sems[:2]
        se = sems[2:]

        def chunk_copies(ci, slot):
            return (
                (row_hbm.at[pl.ds(ebase + ci * ch, ch)], row_c.at[slot]),
                (col_hbm.at[pl.ds(ebase + ci * ch, ch)], col_c.at[slot]),
                (ew_hbm.at[pl.ds(ebase + ci * ch, ch)], ew_c.at[slot]),
            )

        def start_gather(slot, k, b):
            pltpu.async_copy(hp_hbm.at[row_c.at[slot, k]], gbufs[b], sg[b])

        def wait_gather(slot, k, b):
            pltpu.make_async_copy(
                hp_hbm.at[row_c.at[slot, k]], gbufs[b], sg[b]).wait()

        def scale(slot, k, b):
            def grp(t, carry2):
                wvec = ew_c[slot, k, pl.ds(t * 16, 16)]
                base = t * 16
                for l in range(16):
                    wv = jnp.full((16,), wvec[l], jnp.float32)
                    for q in range(F // 16):
                        sl = pl.ds(q * 16, 16)
                        gbufs[b][base + l, sl] = gbufs[b][base + l, sl] * wv
                return carry2

            lax.fori_loop(0, EB // 16, grp, 0)

        for src_ref, dst_ref in chunk_copies(0, 0):
            pltpu.sync_copy(src_ref, dst_ref)
        zbase = s * rows_per_tile
        pltpu.sync_copy(zeros_hbm, acc_s.at[pl.ds(zbase, rows_per_tile)])
        plsc.subcore_barrier()

        for ci in range(nchunk):
            slot = ci % 2
            if ci > 0:
                for src_ref, dst_ref in chunk_copies(ci, slot):
                    pltpu.make_async_copy(src_ref, dst_ref, se[slot]).wait()
            if ci + 1 < nchunk:
                for src_ref, dst_ref in chunk_copies(ci + 1, 1 - slot):
                    pltpu.async_copy(src_ref, dst_ref, se[1 - slot])

            start_gather(slot, 0, 0)

            def pair(i, carry):
                for b in range(2):
                    k = i * 2 + b

                    @pl.when(k + 1 < ch)
                    def _():
                        start_gather(slot, k + 1, 1 - b)

                    wait_gather(slot, k, b)
                    scale(slot, k, b)
                    pltpu.sync_copy(gbufs[b], acc_s.at[col_c.at[slot, k]],
                                    add=True)
                return carry

            lax.fori_loop(0, ch // 2, pair, 0)

        plsc.subcore_barrier()
        pltpu.sync_copy(acc_s.at[pl.ds(zbase, rows_per_tile)],
                        out_hbm.at[pl.ds(c * n_pad + zbase, rows_per_tile)])

    return _spmm_body


def _tc1_body(x_ref, w1_ref, dega_ref, degb_ref, hp_ref, dis_ref):
    deg = dega_ref[...] + degb_ref[...] + 1.0
    dis = jnp.where(deg > 0, lax.rsqrt(deg), 0.0)
    h = jnp.dot(x_ref[...], w1_ref[...], preferred_element_type=jnp.float32)
    hp_ref[...] = h * dis[:, None]
    dis_ref[...] = dis


def _tc2_body(s1a_ref, s1b_ref, hp1_ref, dis_ref, b1_ref, w2_ref, hp2_ref):
    dis = dis_ref[...]
    t = dis[:, None] * (s1a_ref[...] + s1b_ref[...] + hp1_ref[...])
    t = t + b1_ref[...][None, :]
    o = jnp.maximum(t, 0.0)
    h2 = jnp.dot(o, w2_ref[...], preferred_element_type=jnp.float32)
    hp2_ref[...] = h2 * dis[:, None]


def _tc3_body(s2a_ref, s2b_ref, hp2_ref, dis_ref, b2_ref, out_ref):
    dis = dis_ref[...]
    t = dis[:, None] * (s2a_ref[...] + s2b_ref[...] + hp2_ref[...])
    t = t + b2_ref[...][None, :]
    m = jnp.max(t, axis=1, keepdims=True)
    lse = m + jnp.log(jnp.sum(jnp.exp(t - m), axis=1, keepdims=True))
    out_ref[...] = t - lse


def kernel(x, edge_index, edge_weight, W1, b1, W2, b2):
    n, f = x.shape
    e = edge_index.shape[1]
    n_pad = ((n + NW * 8 - 1) // (NW * 8)) * (NW * 8)
    e_quant = NW * EB * 8
    e_pad = ((e + e_quant - 1) // e_quant) * e_quant
    nb_tile = e_pad // (NW * EB)
    rows_per_tile = n_pad // NS

    row = edge_index[0].astype(jnp.int32)
    col = edge_index[1].astype(jnp.int32)
    ew = edge_weight.astype(jnp.float32)
    zi = jnp.zeros((e_pad - e,), jnp.int32)
    row_p = jnp.concatenate([row, zi])
    col_p = jnp.concatenate([col, zi])
    ew_p = jnp.concatenate([ew, jnp.zeros((e_pad - e,), jnp.float32)])
    row2d = row_p.reshape(e_pad // EB, EB)
    col2d = col_p.reshape(e_pad // EB, EB)
    ew2d = ew_p.reshape(e_pad // EB, EB)
    x_p = jnp.concatenate([x, jnp.zeros((n_pad - n, f), x.dtype)], axis=0)
    zeros_rows = jnp.zeros((rows_per_tile, F), jnp.float32)

    deg_k = pl.kernel(
        _deg_body,
        out_type=jax.ShapeDtypeStruct((NC * n_pad,), jnp.float32),
        mesh=_sc_mesh(),
        scratch_types=[
            pltpu.VMEM((nb_tile, EB), jnp.int32),
            pltpu.VMEM((nb_tile, EB), jnp.float32),
            pltpu.VMEM((rows_per_tile,), jnp.float32),
            pltpu.VMEM_SHARED((n_pad,), jnp.float32),
        ],
    )
    deg2 = deg_k(col2d, ew2d)
    deg_a, deg_b = deg2[:n_pad], deg2[n_pad:]

    nchunk = 5
    ch = nb_tile // nchunk
    spmm_k = pl.kernel(
        _make_spmm_body(nb_tile, nchunk),
        out_type=jax.ShapeDtypeStruct((NC * n_pad, F), jnp.float32),
        mesh=_sc_mesh(),
        scratch_types=[
            pltpu.VMEM((2, ch, EB), jnp.int32),
            pltpu.VMEM((2, ch, EB), jnp.int32),
            pltpu.VMEM((2, ch, EB), jnp.float32),
            pltpu.VMEM((EB, F), jnp.float32),
            pltpu.VMEM((EB, F), jnp.float32),
            pltpu.VMEM_SHARED((n_pad, F), jnp.float32),
        ] + [pltpu.SemaphoreType.DMA] * 4,
    )

    def spmm(hp):
        sh = spmm_k(hp, row2d, col2d, ew2d, zeros_rows)
        return sh[:n_pad], sh[n_pad:]

    blk = 1024
    grid = (n_pad // blk,)
    hp1, dis = pl.pallas_call(
        _tc1_body,
        grid=grid,
        in_specs=[
            pl.BlockSpec((blk, f), lambda i: (i, 0)),
            pl.BlockSpec((f, F), lambda i: (0, 0)),
            pl.BlockSpec((blk,), lambda i: (i,)),
            pl.BlockSpec((blk,), lambda i: (i,)),
        ],
        out_specs=[
            pl.BlockSpec((blk, F), lambda i: (i, 0)),
            pl.BlockSpec((blk,), lambda i: (i,)),
        ],
        out_shape=[
            jax.ShapeDtypeStruct((n_pad, F), jnp.float32),
            jax.ShapeDtypeStruct((n_pad,), jnp.float32),
        ],
    )(x_p, W1, deg_a, deg_b)

    s1a, s1b = spmm(hp1)

    hp2 = pl.pallas_call(
        _tc2_body,
        grid=grid,
        in_specs=[
            pl.BlockSpec((blk, F), lambda i: (i, 0)),
            pl.BlockSpec((blk, F), lambda i: (i, 0)),
            pl.BlockSpec((blk, F), lambda i: (i, 0)),
            pl.BlockSpec((blk,), lambda i: (i,)),
            pl.BlockSpec((F,), lambda i: (0,)),
            pl.BlockSpec((F, F), lambda i: (0, 0)),
        ],
        out_specs=pl.BlockSpec((blk, F), lambda i: (i, 0)),
        out_shape=jax.ShapeDtypeStruct((n_pad, F), jnp.float32),
    )(s1a, s1b, hp1, dis, b1, W2)

    s2a, s2b = spmm(hp2)

    out = pl.pallas_call(
        _tc3_body,
        grid=grid,
        in_specs=[
            pl.BlockSpec((blk, F), lambda i: (i, 0)),
            pl.BlockSpec((blk, F), lambda i: (i, 0)),
            pl.BlockSpec((blk, F), lambda i: (i, 0)),
            pl.BlockSpec((blk,), lambda i: (i,)),
            pl.BlockSpec((F,), lambda i: (0,)),
        ],
        out_specs=pl.BlockSpec((blk, F), lambda i: (i, 0)),
        out_shape=jax.ShapeDtypeStruct((n_pad, F), jnp.float32),
    )(s2a, s2b, hp2, dis, b2)

    return out[:n]
